# Initial kernel scaffold; baseline (speedup 1.0000x reference)
import jax
import jax.numpy as jnp
from jax.experimental import pallas as pl


def kernel(input_labels, pos_labels, neg_labels, out_embed_weight):
    raise NotImplementedError("write your pallas kernel here")


# speedup vs baseline: 2.4476x; 2.4476x over previous
import functools

import jax
import jax.numpy as jnp
from jax import lax
from jax.experimental import pallas as pl
from jax.experimental.pallas import tpu as pltpu
from jax.experimental.pallas import tpu_sc as plsc

L = 16
NCTX = 64
E = 2

_LOG2_C = (
    -3.0283249744104577,
    6.065858861121359,
    -5.264155524116715,
    3.218869813800031,
    -1.234279899429953,
    0.26686276780638246,
    -0.024825984442692788,
)
_LN2 = 0.6931471805599453


def _log_f32(z):
    bi = plsc.bitcast(z, jnp.int32)
    ex = lax.shift_right_logical(bi, 23) - 127
    mant = plsc.bitcast(
        lax.bitwise_or(lax.bitwise_and(bi, 0x7FFFFF), 0x3F800000), jnp.float32
    )
    p = jnp.float32(_LOG2_C[6])
    for c in _LOG2_C[5::-1]:
        p = p * mant + jnp.float32(c)
    return (ex.astype(jnp.float32) + p) * jnp.float32(_LN2)


def _make_sc_call(batch, embed, n_pos, n_neg):
    info = plsc.get_sparse_core_info()
    nc, ns = info.num_cores, info.num_subcores
    nw = nc * ns
    epw = batch // nw
    nchunk = epw // E
    n_valid = 1 + n_pos + n_neg

    mesh = plsc.VectorSubcoreMesh(core_axis_name="c", subcore_axis_name="s")

    @functools.partial(
        pl.kernel,
        out_type=jax.ShapeDtypeStruct((batch,), jnp.float32),
        mesh=mesh,
        compiler_params=pltpu.CompilerParams(
            needs_layout_passes=False, use_tc_tiling_on_sc=False
        ),
        scratch_types=[
            pltpu.VMEM((epw * NCTX,), jnp.int32),
            pltpu.VMEM((E * NCTX, embed), jnp.float32),
            pltpu.VMEM((embed,), jnp.float32),
            pltpu.VMEM((epw,), jnp.float32),
            pltpu.SemaphoreType.DMA,
        ],
    )
    def sc_call(table, idxs, out, idx_v, buf, sig_v, out_v, sem):
        iota = lax.iota(jnp.int32, L)
        one = jnp.ones((L,), jnp.float32)
        signs, masks = [], []
        for g in range(4):
            cols = iota + (1 + L * g)
            signs.append(jnp.where(cols <= n_pos, one, -one))
            masks.append(jnp.where(cols < n_valid, one, 0.0 * one))

        wid = lax.axis_index("s") * nc + lax.axis_index("c")
        ibase = wid * (epw * NCTX)
        pltpu.sync_copy(idxs.at[pl.ds(ibase, epw * NCTX)], idx_v)

        def chunk_body(c, carry):
            pltpu.async_copy(
                table.at[idx_v.at[pl.ds(c * (E * NCTX), E * NCTX)]], buf, sem
            ).wait()
            for e in range(E):
                roff = e * NCTX
                for q in range(4):
                    x = buf[roff, pl.ds(L * q, L)]
                    sig_v[pl.ds(L * q, L)] = 1.0 / (1.0 + jnp.exp(-x))
                rows = [
                    jnp.minimum(iota + (roff + 1 + L * g), E * NCTX - 1)
                    for g in range(4)
                ]

                def dbody(j, accs):
                    a0, a1, a2, a3 = accs
                    for k in range(4):
                        d = 4 * j + k
                        col = jnp.full((L,), d, jnp.int32)
                        s = plsc.load_gather(sig_v, [col])
                        a0 = a0 + plsc.load_gather(buf, [rows[0], col]) * s
                        a1 = a1 + plsc.load_gather(buf, [rows[1], col]) * s
                        a2 = a2 + plsc.load_gather(buf, [rows[2], col]) * s
                        a3 = a3 + plsc.load_gather(buf, [rows[3], col]) * s
                    return a0, a1, a2, a3

                zero = jnp.zeros((L,), jnp.float32)
                accs = lax.fori_loop(0, embed // 4, dbody,
                                     (zero, zero, zero, zero))
                total = zero
                for g in range(4):
                    t = accs[g] * signs[g]
                    t = jnp.minimum(jnp.maximum(t, -10.0), 10.0)
                    z = 1.0 + jnp.exp(-t)
                    total = total + _log_f32(z) * masks[g]
                cs = plsc.cumsum(total)
                plsc.store_scatter(
                    out_v,
                    [jnp.full((L,), c * E + e, jnp.int32)],
                    cs,
                    mask=iota == L - 1,
                )
            return carry

        lax.fori_loop(0, nchunk, chunk_body, 0)
        pltpu.sync_copy(out_v, out.at[pl.ds(wid * epw, epw)])

    return sc_call


def kernel(input_labels, pos_labels, neg_labels, out_embed_weight):
    batch = input_labels.shape[0]
    n_pos = pos_labels.shape[1]
    n_neg = neg_labels.shape[1]
    embed = out_embed_weight.shape[1]
    pad = NCTX - (1 + n_pos + n_neg)
    idx = jnp.concatenate(
        [
            input_labels[:, None],
            pos_labels,
            neg_labels,
            jnp.zeros((batch, pad), input_labels.dtype),
        ],
        axis=1,
    ).astype(jnp.int32).reshape(-1)
    sc_call = _make_sc_call(batch, embed, n_pos, n_neg)
    return sc_call(out_embed_weight, idx)
